# baseline (speedup 1.0000x reference)
import functools

import jax
import jax.numpy as jnp
from jax import lax
from jax.experimental import pallas as pl
from jax.experimental.pallas import tpu as pltpu
from jax.experimental.pallas import tpu_sc as plsc


def _sc_cat_partials(tbl, stdp, x_t, y_t, B, F_CAT, NCAT, RANK, NPAD):
    info = plsc.get_sparse_core_info()
    NC, NS, L = info.num_cores, info.num_subcores, info.num_lanes
    NW = NC * NS
    NPAIR = B * F_CAT
    P = NPAIR // NW
    W = RANK // 2
    FS = NPAD * W
    groups = P // L

    PR = P // L
    BR = B // L

    @functools.partial(
        pl.kernel,
        out_type=jax.ShapeDtypeStruct((NC, BR, L), jnp.float32),
        mesh=plsc.VectorSubcoreMesh(core_axis_name="c", subcore_axis_name="s"),
        compiler_params=pltpu.CompilerParams(
            needs_layout_passes=False, use_tc_tiling_on_sc=False
        ),
        scratch_types=[
            pltpu.VMEM((2 * FS,), jnp.int32),
            pltpu.VMEM((2 * NPAD,), jnp.float32),
            pltpu.VMEM((P,), jnp.int32),
            pltpu.VMEM((P,), jnp.int32),
            pltpu.VMEM((PR, L), jnp.float32),
            pltpu.VMEM((PR,), jnp.int32),
            pltpu.VMEM((BR, L), jnp.float32),
            pltpu.VMEM_SHARED((BR, L), jnp.float32),
        ],
    )
    def k(tbl_hbm, std_hbm, x_hbm, y_hbm, out_hbm,
          tbl_v, std_v, x_v, y_v, out_v, idx_v, zero_v, acc_sh):
        c = lax.axis_index("c")
        s = lax.axis_index("s")
        wid = s * NC + c
        q0 = wid * P
        f0 = q0 // B
        pltpu.sync_copy(tbl_hbm.at[pl.ds(f0 * FS, 2 * FS)], tbl_v)
        pltpu.sync_copy(std_hbm.at[pl.ds(f0 * NPAD, 2 * NPAD)], std_v)
        pltpu.sync_copy(x_hbm.at[pl.ds(q0, P)], x_v)
        pltpu.sync_copy(y_hbm.at[pl.ds(q0, P)], y_v)

        def group(g, carry):
            base = g * L
            df = (q0 + base) // B - f0
            toff = jnp.full((L,), df * FS, jnp.int32)
            soff = jnp.full((L,), df * NPAD, jnp.int32)
            bx = x_v[pl.ds(base, L)]
            by = y_v[pl.ds(base, L)]
            bxt = bx | toff
            byt = by | toff
            acc = jnp.zeros((L,), jnp.float32)
            for w in range(W):
                ix = plsc.load_gather(tbl_v, [bxt | (w * NPAD)])
                iy = plsc.load_gather(tbl_v, [byt | (w * NPAD)])
                pr = plsc.bitcast(ix, jnp.bfloat16) * plsc.bitcast(
                    iy, jnp.bfloat16
                )
                p0, p1 = plsc.unpack(pr, format=plsc.PackFormat.INTERLEAVED)
                acc = acc + (p0 + p1)
            sv = plsc.load_gather(std_v, [bx | soff])
            acc = acc + jnp.where(bx == by, sv * sv, jnp.zeros((L,), jnp.float32))
            out_v[g] = acc
            return carry

        lax.fori_loop(0, groups, group, 0)

        iot = lax.iota(jnp.int32, L)

        def mkidx(j, carry):
            idx_v[pl.ds(j * L, L)] = (iot + (q0 // L + j * L)) & (BR - 1)
            return carry

        lax.fori_loop(0, PR // L, mkidx, 0)

        @pl.when(s == 0)
        def _():
            def zrow(i, carry):
                zero_v[i] = jnp.zeros((L,), jnp.float32)
                return carry

            lax.fori_loop(0, BR, zrow, 0)
            pltpu.sync_copy(zero_v, acc_sh)

        plsc.subcore_barrier()
        pltpu.sync_copy(out_v, acc_sh.at[idx_v], add=True)
        plsc.subcore_barrier()

        @pl.when(s == 0)
        def _():
            pltpu.sync_copy(acc_sh, out_hbm.at[c])

    return k(tbl, stdp, x_t, y_t).reshape(-1), NC


def _tc_cont(xc_t, yc_t, bw, ss, B, F_CONT):
    BLK = 2048

    def body(x_ref, y_ref, bw_ref, ss_ref, o_ref):
        d = x_ref[...] - y_ref[...]
        bwv = bw_ref[...]
        inv = 1.0 / (2.0 * bwv * bwv)
        scale = ss_ref[...] * ss_ref[...]
        o_ref[...] = jnp.sum(scale * jnp.exp(-(d * d) * inv), axis=0)

    return pl.pallas_call(
        body,
        grid=(B // BLK,),
        in_specs=[
            pl.BlockSpec((F_CONT, BLK), lambda i: (0, i)),
            pl.BlockSpec((F_CONT, BLK), lambda i: (0, i)),
            pl.BlockSpec((F_CONT, 1), lambda i: (0, 0)),
            pl.BlockSpec((F_CONT, 1), lambda i: (0, 0)),
        ],
        out_specs=pl.BlockSpec((BLK,), lambda i: (i,)),
        out_shape=jax.ShapeDtypeStruct((B,), jnp.float32),
    )(xc_t, yc_t, bw, ss)


def _tc_combine(partials_flat, cont, B, NROW):
    BLK = 2048
    nb = B // BLK

    def body(*refs):
        p_refs = refs[: NROW + 1]
        o_ref = refs[NROW + 1]
        acc = p_refs[0][...]
        for pr in p_refs[1:]:
            acc = acc + pr[...]
        o_ref[...] = acc

    cat_specs = [pl.BlockSpec((BLK,), lambda i: (i,))] + [
        pl.BlockSpec((BLK,), (lambda r: (lambda i: (r * nb + i,)))(r))
        for r in range(NROW)
    ]
    return pl.pallas_call(
        body,
        grid=(nb,),
        in_specs=cat_specs,
        out_specs=pl.BlockSpec((BLK,), lambda i: (i,)),
        out_shape=jax.ShapeDtypeStruct((B,), jnp.float32),
    )(cont, *([partials_flat] * NROW))


def kernel(x_cat, x_cont, y_cat, y_cont, bandwidth, sqrt_scale, std, covar_factor):
    B, F_CAT = x_cat.shape
    F_CONT = x_cont.shape[1]
    NCAT = std.shape[1]
    RANK = covar_factor.shape[2]
    NPAD = NCAT + (-NCAT) % 1024
    cf_b = covar_factor.transpose(0, 2, 1).astype(jnp.bfloat16)
    lo = lax.bitcast_convert_type(cf_b[:, 0::2, :], jnp.uint16).astype(
        jnp.int32
    )
    hi = lax.bitcast_convert_type(cf_b[:, 1::2, :], jnp.uint16).astype(
        jnp.int32
    )
    tbl = jnp.pad(
        lo | (hi << 16), ((0, 1), (0, 0), (0, NPAD - NCAT))
    ).reshape(-1)
    stdp = jnp.pad(std, ((0, 1), (0, NPAD - NCAT))).reshape(-1)
    partials, nrow = _sc_cat_partials(
        tbl,
        stdp,
        x_cat.T.reshape(-1),
        y_cat.T.reshape(-1),
        B,
        F_CAT,
        NCAT,
        RANK,
        NPAD,
    )
    cont = _tc_cont(
        x_cont.T,
        y_cont.T,
        bandwidth.reshape(F_CONT, 1),
        sqrt_scale.reshape(F_CONT, 1),
        B,
        F_CONT,
    )
    return _tc_combine(partials, cont, B, nrow)


# speedup vs baseline: 1.1123x; 1.1123x over previous
import functools

import jax
import jax.numpy as jnp
from jax import lax
from jax.experimental import pallas as pl
from jax.experimental.pallas import tpu as pltpu
from jax.experimental.pallas import tpu_sc as plsc


def _sc_cat_partials(tbl, stdp, x_t, y_t, B, F_CAT, NCAT, RANK, NPAD):
    info = plsc.get_sparse_core_info()
    NC, NS, L = info.num_cores, info.num_subcores, info.num_lanes
    NW = NC * NS
    NPAIR = B * F_CAT
    P = NPAIR // NW
    W = RANK // 2
    FS = NPAD * W
    groups = P // L

    @functools.partial(
        pl.kernel,
        out_type=jax.ShapeDtypeStruct((NPAIR,), jnp.float32),
        mesh=plsc.VectorSubcoreMesh(core_axis_name="c", subcore_axis_name="s"),
        compiler_params=pltpu.CompilerParams(
            needs_layout_passes=False, use_tc_tiling_on_sc=False
        ),
        scratch_types=[
            pltpu.VMEM((2 * FS,), jnp.int32),
            pltpu.VMEM((2 * NPAD,), jnp.float32),
            pltpu.VMEM((P,), jnp.int32),
            pltpu.VMEM((P,), jnp.int32),
            pltpu.VMEM((P,), jnp.float32),
        ],
    )
    def k(tbl_hbm, std_hbm, x_hbm, y_hbm, out_hbm, tbl_v, std_v, x_v, y_v, out_v):
        c = lax.axis_index("c")
        s = lax.axis_index("s")
        wid = s * NC + c
        q0 = wid * P
        f0 = q0 // B
        pltpu.sync_copy(tbl_hbm.at[pl.ds(f0 * FS, 2 * FS)], tbl_v)
        pltpu.sync_copy(std_hbm.at[pl.ds(f0 * NPAD, 2 * NPAD)], std_v)
        pltpu.sync_copy(x_hbm.at[pl.ds(q0, P)], x_v)
        pltpu.sync_copy(y_hbm.at[pl.ds(q0, P)], y_v)

        def group(g, carry):
            base = g * L
            df = (q0 + base) // B - f0
            toff = jnp.full((L,), df * FS, jnp.int32)
            soff = jnp.full((L,), df * NPAD, jnp.int32)
            bx = x_v[pl.ds(base, L)]
            by = y_v[pl.ds(base, L)]
            bxt = bx | toff
            byt = by | toff
            acc = jnp.zeros((L,), jnp.float32)
            for w in range(W):
                ix = plsc.load_gather(tbl_v, [bxt | (w * NPAD)])
                iy = plsc.load_gather(tbl_v, [byt | (w * NPAD)])
                pr = plsc.bitcast(ix, jnp.bfloat16) * plsc.bitcast(
                    iy, jnp.bfloat16
                )
                p0, p1 = plsc.unpack(pr, format=plsc.PackFormat.INTERLEAVED)
                acc = acc + (p0 + p1)
            sv = plsc.load_gather(std_v, [bx | soff])
            acc = acc + jnp.where(bx == by, sv * sv, jnp.zeros((L,), jnp.float32))
            out_v[pl.ds(base, L)] = acc
            return carry

        lax.fori_loop(0, groups, group, 0)
        pltpu.sync_copy(out_v, out_hbm.at[pl.ds(q0, P)])

    return k(tbl, stdp, x_t, y_t)


def _tc_cont(xc_t, yc_t, bw, ss, B, F_CONT):
    BLK = 2048

    def body(x_ref, y_ref, bw_ref, ss_ref, o_ref):
        d = x_ref[...] - y_ref[...]
        bwv = bw_ref[...]
        inv = 1.0 / (2.0 * bwv * bwv)
        scale = ss_ref[...] * ss_ref[...]
        o_ref[...] = jnp.sum(scale * jnp.exp(-(d * d) * inv), axis=0)

    return pl.pallas_call(
        body,
        grid=(B // BLK,),
        in_specs=[
            pl.BlockSpec((F_CONT, BLK), lambda i: (0, i)),
            pl.BlockSpec((F_CONT, BLK), lambda i: (0, i)),
            pl.BlockSpec((F_CONT, 1), lambda i: (0, 0)),
            pl.BlockSpec((F_CONT, 1), lambda i: (0, 0)),
        ],
        out_specs=pl.BlockSpec((BLK,), lambda i: (i,)),
        out_shape=jax.ShapeDtypeStruct((B,), jnp.float32),
    )(xc_t, yc_t, bw, ss)


def _tc_combine(partials_flat, cont, B, F_CAT):
    BLK = 4096
    nb = B // BLK

    def body(*refs):
        p_refs = refs[: F_CAT + 1]
        o_ref = refs[F_CAT + 1]
        acc = p_refs[0][...]
        for pr in p_refs[1:]:
            acc = acc + pr[...]
        o_ref[...] = acc

    cat_specs = [pl.BlockSpec((BLK,), lambda i: (i,))] + [
        pl.BlockSpec((BLK,), (lambda f: (lambda i: (f * nb + i,)))(f))
        for f in range(F_CAT)
    ]
    return pl.pallas_call(
        body,
        grid=(nb,),
        in_specs=cat_specs,
        out_specs=pl.BlockSpec((BLK,), lambda i: (i,)),
        out_shape=jax.ShapeDtypeStruct((B,), jnp.float32),
    )(cont, *([partials_flat] * F_CAT))


def kernel(x_cat, x_cont, y_cat, y_cont, bandwidth, sqrt_scale, std, covar_factor):
    B, F_CAT = x_cat.shape
    F_CONT = x_cont.shape[1]
    NCAT = std.shape[1]
    RANK = covar_factor.shape[2]
    NPAD = NCAT + (-NCAT) % 1024
    cf_b = covar_factor.transpose(0, 2, 1).astype(jnp.bfloat16)
    lo = lax.bitcast_convert_type(cf_b[:, 0::2, :], jnp.uint16).astype(
        jnp.int32
    )
    hi = lax.bitcast_convert_type(cf_b[:, 1::2, :], jnp.uint16).astype(
        jnp.int32
    )
    tbl = jnp.pad(
        lo | (hi << 16), ((0, 1), (0, 0), (0, NPAD - NCAT))
    ).reshape(-1)
    stdp = jnp.pad(std, ((0, 1), (0, NPAD - NCAT))).reshape(-1)
    partials = _sc_cat_partials(
        tbl,
        stdp,
        x_cat.T.reshape(-1),
        y_cat.T.reshape(-1),
        B,
        F_CAT,
        NCAT,
        RANK,
        NPAD,
    )
    cont = _tc_cont(
        x_cont.T,
        y_cont.T,
        bandwidth.reshape(F_CONT, 1),
        sqrt_scale.reshape(F_CONT, 1),
        B,
        F_CONT,
    )
    return _tc_combine(partials, cont, B, F_CAT)


# speedup vs baseline: 1.1408x; 1.0256x over previous
import functools

import jax
import jax.numpy as jnp
from jax import lax
from jax.experimental import pallas as pl
from jax.experimental.pallas import tpu as pltpu
from jax.experimental.pallas import tpu_sc as plsc


def _sc_cat_partials(tbl, stdp, x_t, y_t, B, F_CAT, NCAT, RANK, NPAD):
    info = plsc.get_sparse_core_info()
    NC, NS, L = info.num_cores, info.num_subcores, info.num_lanes
    NW = NC * NS
    NPAIR = B * F_CAT
    P = NPAIR // NW
    W = RANK // 2
    FS = NPAD * W
    groups = P // L

    @functools.partial(
        pl.kernel,
        out_type=jax.ShapeDtypeStruct((NPAIR,), jnp.float32),
        mesh=plsc.VectorSubcoreMesh(core_axis_name="c", subcore_axis_name="s"),
        compiler_params=pltpu.CompilerParams(
            needs_layout_passes=False, use_tc_tiling_on_sc=False
        ),
        scratch_types=[
            pltpu.VMEM((2 * FS,), jnp.int32),
            pltpu.VMEM((2 * NPAD,), jnp.float32),
            pltpu.VMEM((P,), jnp.int32),
            pltpu.VMEM((P,), jnp.int32),
            pltpu.VMEM((P,), jnp.float32),
        ],
    )
    def k(tbl_hbm, std_hbm, x_hbm, y_hbm, out_hbm, tbl_v, std_v, x_v, y_v, out_v):
        c = lax.axis_index("c")
        s = lax.axis_index("s")
        wid = s * NC + c
        q0 = wid * P
        f0 = q0 // B
        pltpu.sync_copy(tbl_hbm.at[pl.ds(f0 * FS, 2 * FS)], tbl_v)
        pltpu.sync_copy(std_hbm.at[pl.ds(f0 * NPAD, 2 * NPAD)], std_v)
        pltpu.sync_copy(x_hbm.at[pl.ds(q0, P)], x_v)
        pltpu.sync_copy(y_hbm.at[pl.ds(q0, P)], y_v)

        def group(g, carry):
            base = g * L
            df = (q0 + base) // B - f0
            toff = jnp.full((L,), df * FS, jnp.int32)
            soff = jnp.full((L,), df * NPAD, jnp.int32)
            bx = x_v[pl.ds(base, L)]
            by = y_v[pl.ds(base, L)]
            bxt = bx | toff
            byt = by | toff
            acc = jnp.zeros((L,), jnp.float32)
            for w in range(W):
                ix = plsc.load_gather(tbl_v, [bxt | (w * NPAD)])
                iy = plsc.load_gather(tbl_v, [byt | (w * NPAD)])
                pr = plsc.bitcast(ix, jnp.bfloat16) * plsc.bitcast(
                    iy, jnp.bfloat16
                )
                p0, p1 = plsc.unpack(pr, format=plsc.PackFormat.INTERLEAVED)
                acc = acc + (p0 + p1)
            sv = plsc.load_gather(std_v, [bx | soff])
            acc = acc + jnp.where(bx == by, sv * sv, jnp.zeros((L,), jnp.float32))
            out_v[pl.ds(base, L)] = acc
            return carry

        lax.fori_loop(0, groups, group, 0)
        pltpu.sync_copy(out_v, out_hbm.at[pl.ds(q0, P)])

    return k(tbl, stdp, x_t, y_t)


def _tc_cont(xc_t, yc_t, bw, ss, B, F_CONT):
    BLK = 2048

    def body(x_ref, y_ref, bw_ref, ss_ref, o_ref):
        d = x_ref[...] - y_ref[...]
        bwv = bw_ref[...]
        inv = 1.0 / (2.0 * bwv * bwv)
        scale = ss_ref[...] * ss_ref[...]
        o_ref[...] = jnp.sum(scale * jnp.exp(-(d * d) * inv), axis=0)

    return pl.pallas_call(
        body,
        grid=(B // BLK,),
        in_specs=[
            pl.BlockSpec((F_CONT, BLK), lambda i: (0, i)),
            pl.BlockSpec((F_CONT, BLK), lambda i: (0, i)),
            pl.BlockSpec((F_CONT, 1), lambda i: (0, 0)),
            pl.BlockSpec((F_CONT, 1), lambda i: (0, 0)),
        ],
        out_specs=pl.BlockSpec((BLK,), lambda i: (i,)),
        out_shape=jax.ShapeDtypeStruct((B,), jnp.float32),
    )(xc_t, yc_t, bw, ss)


def _tc_combine(partials_flat, cont, B, F_CAT):
    BLK = 8192
    nb = B // BLK

    def body(*refs):
        p_refs = refs[: F_CAT + 1]
        o_ref = refs[F_CAT + 1]
        acc = p_refs[0][...]
        for pr in p_refs[1:]:
            acc = acc + pr[...]
        o_ref[...] = acc

    cat_specs = [pl.BlockSpec((BLK,), lambda i: (i,))] + [
        pl.BlockSpec((BLK,), (lambda f: (lambda i: (f * nb + i,)))(f))
        for f in range(F_CAT)
    ]
    return pl.pallas_call(
        body,
        grid=(nb,),
        in_specs=cat_specs,
        out_specs=pl.BlockSpec((BLK,), lambda i: (i,)),
        out_shape=jax.ShapeDtypeStruct((B,), jnp.float32),
    )(cont, *([partials_flat] * F_CAT))


def kernel(x_cat, x_cont, y_cat, y_cont, bandwidth, sqrt_scale, std, covar_factor):
    B, F_CAT = x_cat.shape
    F_CONT = x_cont.shape[1]
    NCAT = std.shape[1]
    RANK = covar_factor.shape[2]
    NPAD = NCAT + (-NCAT) % 1024
    cf_b = covar_factor.transpose(0, 2, 1).astype(jnp.bfloat16)
    lo = lax.bitcast_convert_type(cf_b[:, 0::2, :], jnp.uint16).astype(
        jnp.int32
    )
    hi = lax.bitcast_convert_type(cf_b[:, 1::2, :], jnp.uint16).astype(
        jnp.int32
    )
    tbl = jnp.pad(
        lo | (hi << 16), ((0, 1), (0, 0), (0, NPAD - NCAT))
    ).reshape(-1)
    stdp = jnp.pad(std, ((0, 1), (0, NPAD - NCAT))).reshape(-1)
    partials = _sc_cat_partials(
        tbl,
        stdp,
        x_cat.T.reshape(-1),
        y_cat.T.reshape(-1),
        B,
        F_CAT,
        NCAT,
        RANK,
        NPAD,
    )
    cont = _tc_cont(
        x_cont.T,
        y_cont.T,
        bandwidth.reshape(F_CONT, 1),
        sqrt_scale.reshape(F_CONT, 1),
        B,
        F_CONT,
    )
    return _tc_combine(partials, cont, B, F_CAT)
